# baseline (speedup 1.0000x reference)
import functools

import jax
import jax.numpy as jnp
from jax import lax
from jax.experimental import pallas as pl
from jax.experimental.pallas import tpu as pltpu

_NBUF = 5


def _copy_in(x_hbm, buf, sems, step, tile, nsteps):
    @pl.when(step < nsteps)
    def _():
        slot = lax.rem(step, _NBUF)
        pltpu.make_async_copy(
            x_hbm.at[pl.ds(step * tile, tile), :],
            buf.at[slot],
            sems.at[slot],
        ).start()


def _fused_heads_kernel(x_hbm, wc_ref, wp_ref, bc_ref, bp_ref,
                        coords_ref, pres_ref, buf, sems, *, tile, nsteps):
    i = pl.program_id(0)

    @pl.when(i == 0)
    def _():
        for j in range(min(_NBUF - 1, nsteps)):
            pltpu.make_async_copy(
                x_hbm.at[pl.ds(j * tile, tile), :],
                buf.at[j],
                sems.at[j],
            ).start()

    _copy_in(x_hbm, buf, sems, i + _NBUF - 1, tile, nsteps)

    slot = lax.rem(i, _NBUF)
    pltpu.make_async_copy(
        x_hbm.at[pl.ds(i * tile, tile), :],
        buf.at[slot],
        sems.at[slot],
    ).wait()

    x = buf[slot]
    w = jnp.concatenate([wc_ref[...], wp_ref[...]], axis=0)
    y = lax.dot_general(
        x, w,
        dimension_numbers=(((1,), (1,)), ((), ())),
        preferred_element_type=jnp.float32,
    )
    coords_ref[...] = y[:, 0:4] + bc_ref[...]
    pres_ref[...] = y[:, 4:5] + bp_ref[...]


@functools.partial(jax.jit, static_argnames=("tile",))
def _run(x, wc, wp, bc, bp, tile):
    n, d = x.shape
    nsteps = n // tile
    body = functools.partial(_fused_heads_kernel, tile=tile, nsteps=nsteps)
    coords, pres = pl.pallas_call(
        body,
        grid=(nsteps,),
        in_specs=[
            pl.BlockSpec(memory_space=pl.ANY),
            pl.BlockSpec(wc.shape, lambda i: (0, 0)),
            pl.BlockSpec(wp.shape, lambda i: (0, 0)),
            pl.BlockSpec(bc.shape, lambda i: (0, 0)),
            pl.BlockSpec(bp.shape, lambda i: (0, 0)),
        ],
        out_specs=[
            pl.BlockSpec((tile, 4), lambda i: (i, 0)),
            pl.BlockSpec((tile, 1), lambda i: (i, 0)),
        ],
        out_shape=[
            jax.ShapeDtypeStruct((n, 4), jnp.float32),
            jax.ShapeDtypeStruct((n, 1), jnp.float32),
        ],
        scratch_shapes=[
            pltpu.VMEM((_NBUF, tile, d), jnp.float32),
            pltpu.SemaphoreType.DMA((_NBUF,)),
        ],
        compiler_params=pltpu.CompilerParams(
            dimension_semantics=("arbitrary",),
        ),
    )(x, wc, wp, bc, bp)
    return coords, pres


def kernel(local_features, W_coords, b_coords, W_pres, b_pres):
    B, C, R, D = local_features.shape
    n = B * C * R
    x = local_features.reshape(n, D)
    coords, pres = _run(
        x, W_coords, W_pres,
        b_coords.reshape(1, 4), b_pres.reshape(1, 1),
        2000,
    )
    return (
        coords.reshape(B, C, R, 4),
        pres.reshape(B, C, R, 1),
    )
